# baseline (speedup 1.0000x reference)
import functools

import jax
import jax.numpy as jnp
from jax import lax
from jax.experimental import pallas as pl
from jax.experimental.pallas import tpu as pltpu
from jax.experimental.pallas import tpu_sc as plsc

D = 64
L = 16
NC, NS = 2, 16
NW = NC * NS
S0 = 4096
S1 = 50
CH = 128
PW = S1 * CH
R = 5
AH = 3
W = 2

_mesh = plsc.VectorSubcoreMesh(core_axis_name="c", subcore_axis_name="s")


@functools.partial(
    pl.kernel,
    out_type=jax.ShapeDtypeStruct((S1, D, S0), jnp.float32),
    mesh=_mesh,
    scratch_types=(
        [
            pltpu.VMEM((PW,), jnp.int32),
            pltpu.VMEM((S1, CH), jnp.int32),
            pltpu.VMEM((S1, CH), jnp.int32),
            pltpu.VMEM((R, CH, 2 * D), jnp.float32),
            pltpu.VMEM((W, D, CH), jnp.float32),
        ]
        + [pltpu.SemaphoreType.DMA] * (R + W)
    ),
    compiler_params=pltpu.CompilerParams(
        use_tc_tiling_on_sc=True, needs_layout_passes=False
    ),
)
def _lookup(tok_hbm, table2_hbm, out_hbm, idx_v, pidx_v, ppar_v, gbuf, tbuf, *sems):
    gsem, wsem = sems[:R], sems[R:]
    wid = lax.axis_index("s") * NC + lax.axis_index("c")
    base = wid * PW
    pltpu.sync_copy(tok_hbm.at[pl.ds(base, PW)], idx_v)

    lanes = lax.iota(jnp.int32, L)
    l50 = lanes * S1
    one = jnp.ones((L,), jnp.int32)

    @pl.loop(0, S1)
    def _panelize(s):
        for b16 in range(CH // L):
            pos = l50 + (s + b16 * (L * S1))
            v = plsc.load_gather(idx_v, [pos])
            pidx_v[s, pl.ds(b16 * L, L)] = lax.shift_right_logical(v, 1)
            ppar_v[s, pl.ds(b16 * L, L)] = lax.bitwise_and(v, one) * D

    def _fire_gather(s, b):
        pltpu.async_copy(
            table2_hbm.at[pidx_v.at[s]], gbuf.at[b], gsem[b]
        )

    for b in range(AH):
        _fire_gather(b, b)

    @pl.loop(0, S1, step=R)
    def _group(g):
        for b in range(R):
            s = g + b
            w = b % W
            pltpu.make_async_copy(
                table2_hbm.at[pl.ds(0, CH)], gbuf.at[b], gsem[b]
            ).wait()
            nxt = s + AH

            @pl.when(nxt < S1)
            def _fire():
                _fire_gather(nxt, (b + AH) % R)

            @pl.when(s - W >= 0)
            def _drain_write():
                pltpu.make_async_copy(
                    tbuf.at[w], out_hbm.at[0, :, pl.ds(0, CH)], wsem[w]
                ).wait()

            bvec = jnp.full((L,), b, jnp.int32)
            for b16 in range(CH // L):
                rows = lanes + b16 * L
                par = ppar_v[s, pl.ds(b16 * L, L)]
                for d in range(D):
                    vd = plsc.load_gather(gbuf, [bvec, rows, par + d])
                    tbuf[w, d, pl.ds(b16 * L, L)] = vd
            pltpu.async_copy(
                tbuf.at[w], out_hbm.at[s, :, pl.ds(wid * CH, CH)], wsem[w]
            )

    for w in range(W):
        pltpu.make_async_copy(
            tbuf.at[w], out_hbm.at[0, :, pl.ds(0, CH)], wsem[w]
        ).wait()


def kernel(tokens, table):
    tok = tokens.reshape(S0 * S1).astype(jnp.int32)
    table2 = table.reshape(500000, 2 * D)
    out3 = _lookup(tok, table2)
    return out3.transpose(2, 0, 1)


# speedup vs baseline: 1.2079x; 1.2079x over previous
import functools

import jax
import jax.numpy as jnp
from jax import lax
from jax.experimental import pallas as pl
from jax.experimental.pallas import tpu as pltpu
from jax.experimental.pallas import tpu_sc as plsc

D = 64
NC, NS = 2, 16
NW = NC * NS
CH = 128
NCH = 50
PW = NCH * CH
B = NW * PW
R = 10
AH = 5

_mesh = plsc.VectorSubcoreMesh(core_axis_name="c", subcore_axis_name="s")


@functools.partial(
    pl.kernel,
    out_type=jax.ShapeDtypeStruct((B, D), jnp.float32),
    mesh=_mesh,
    scratch_types=(
        [pltpu.VMEM((PW,), jnp.int32), pltpu.VMEM((R, CH, D), jnp.float32)]
        + [pltpu.SemaphoreType.DMA] * (2 * R)
    ),
    compiler_params=pltpu.CompilerParams(use_tc_tiling_on_sc=False),
)
def _lookup(tok_hbm, table_hbm, out_hbm, idx_v, rows_v, *sems):
    gsem, wsem = sems[:R], sems[R:]
    wid = lax.axis_index("s") * NC + lax.axis_index("c")
    base = wid * PW
    pltpu.sync_copy(tok_hbm.at[pl.ds(base, PW)], idx_v)

    for b in range(AH):
        pltpu.async_copy(
            table_hbm.at[idx_v.at[pl.ds(b * CH, CH)]], rows_v.at[b], gsem[b]
        )

    @pl.loop(0, NCH, step=R)
    def _group(g):
        for b in range(R):
            j = g + b
            pltpu.make_async_copy(
                table_hbm.at[pl.ds(0, CH)], rows_v.at[b], gsem[b]
            ).wait()
            pltpu.async_copy(
                rows_v.at[b], out_hbm.at[pl.ds(base + j * CH, CH)], wsem[b]
            )
            nb = (b + AH) % R
            nxt = j + AH

            @pl.when(nxt < NCH)
            def _fire():
                @pl.when(nxt - R >= 0)
                def _drain_old_write():
                    pltpu.make_async_copy(
                        rows_v.at[nb], out_hbm.at[pl.ds(base, CH)], wsem[nb]
                    ).wait()

                pltpu.async_copy(
                    table_hbm.at[idx_v.at[pl.ds(nxt * CH, CH)]],
                    rows_v.at[nb],
                    gsem[nb],
                )

    for b in range(R):
        pltpu.make_async_copy(
            rows_v.at[b], out_hbm.at[pl.ds(base, CH)], wsem[b]
        ).wait()


def kernel(tokens, table):
    s0, s1 = tokens.shape
    tok = tokens.reshape(B).astype(jnp.int32)
    out = _lookup(tok, table)
    return out.reshape(s0, s1, D)
